# baseline (speedup 1.0000x reference)
import functools

import jax
import jax.numpy as jnp
import numpy as np
from jax import lax
from jax.experimental import pallas as pl
from jax.experimental.pallas import tpu as pltpu
from jax.experimental.pallas import tpu_sc as plsc

N = 50000
D = 128
H = 4
DH = 32
K = 8
FF = 512
OUT = 128

NW = 32
NP = 50176
PW = NP // NW
IPW = PW * K
GCH = 128
NCHUNK = IPW // GCH

BLK = 512
ABLK = 256


def _lnorm(h, s, b):
    m = jnp.mean(h, axis=-1, keepdims=True)
    v = jnp.mean((h - m) ** 2, axis=-1, keepdims=True)
    return (h - m) * lax.rsqrt(v + 1e-5) * s + b


def _row_spec(blk, width):
    return pl.BlockSpec((blk, width), lambda i: (i, 0))


def _full_spec(shape):
    return pl.BlockSpec(shape, lambda i: tuple(0 for _ in shape))


def _pack_kv(k, v):
    kw = lax.bitcast_convert_type(k.astype(jnp.bfloat16), jnp.uint16).astype(jnp.uint32)
    vw = lax.bitcast_convert_type(v.astype(jnp.bfloat16), jnp.uint16).astype(jnp.uint32)
    return ((kw << 16) | vw).astype(jnp.int32)


def _unpack_kv(w):
    ww = lax.bitcast_convert_type(w, jnp.uint32)
    k = lax.bitcast_convert_type(ww & jnp.uint32(0xFFFF0000), jnp.float32)
    v = lax.bitcast_convert_type(ww << 16, jnp.float32)
    return k, v


def _qkv_part(h, ls, lb, wq, bq, wk, bk, wv, bv):
    hn = _lnorm(h, ls[...], lb[...])
    q = hn @ wq[...] + bq[...]
    k = hn @ wk[...] + bk[...]
    v = hn @ wv[...] + bv[...]
    return q, _pack_kv(k, v)


def _enc_qkv_body(x_ref, w1, b1, w2, b2, els, elb,
                  ls, lb, wq, bq, wk, bk, wv, bv,
                  h_ref, q_ref, kv_ref):
    h = jnp.maximum(x_ref[...] @ w1[...] + b1[...], 0.0)
    h = h @ w2[...] + b2[...]
    h = _lnorm(h, els[...], elb[...])
    h_ref[...] = h
    q, kv = _qkv_part(h, ls, lb, wq, bq, wk, bk, wv, bv)
    q_ref[...] = q
    kv_ref[...] = kv


def _enc_qkv(x, e, p):
    grid = (NP // BLK,)
    shp = jax.ShapeDtypeStruct((NP, D), jnp.float32)
    shi = jax.ShapeDtypeStruct((NP, D), jnp.int32)
    return pl.pallas_call(
        _enc_qkv_body,
        grid=grid,
        in_specs=[
            _row_spec(BLK, D),
            _full_spec((D, D)), _full_spec((1, D)),
            _full_spec((D, D)), _full_spec((1, D)),
            _full_spec((1, D)), _full_spec((1, D)),
            _full_spec((1, D)), _full_spec((1, D)),
            _full_spec((D, D)), _full_spec((1, D)),
            _full_spec((D, D)), _full_spec((1, D)),
            _full_spec((D, D)), _full_spec((1, D)),
        ],
        out_specs=[_row_spec(BLK, D)] * 3,
        out_shape=[shp, shp, shi],
    )(x, e['W1'], e['b1'].reshape(1, D), e['W2'], e['b2'].reshape(1, D),
      e['ln_s'].reshape(1, D), e['ln_b'].reshape(1, D),
      p['ln1_s'].reshape(1, D), p['ln1_b'].reshape(1, D),
      p['Wq'], p['bq'].reshape(1, D), p['Wk'], p['bk'].reshape(1, D),
      p['Wv'], p['bv'].reshape(1, D))


def _attn_body(q_ref, kvs_ref, kvg_ref, o_ref):
    q = q_ref[...]
    r = lax.broadcasted_iota(jnp.int32, (D, H), 0) // DH
    c = lax.broadcasted_iota(jnp.int32, (D, H), 1)
    S = (r == c).astype(jnp.float32)
    r2 = lax.broadcasted_iota(jnp.int32, (H, D), 0)
    c2 = lax.broadcasted_iota(jnp.int32, (H, D), 1) // DH
    ST = (r2 == c2).astype(jnp.float32)
    scale = np.float32(1.0 / np.sqrt(DH))
    logit = []
    vals = []
    for j in range(K + 1):
        kj, vj = _unpack_kv(kvs_ref[...] if j == 0 else kvg_ref[j - 1])
        logit.append(((q * kj) @ S) * scale)
        vals.append(vj)
    m = logit[0]
    for x in logit[1:]:
        m = jnp.maximum(m, x)
    es = [jnp.exp(x - m) for x in logit]
    ssum = es[0]
    for e in es[1:]:
        ssum = ssum + e
    rinv = 1.0 / ssum
    o = ((es[0] * rinv) @ ST) * vals[0]
    for j in range(1, K + 1):
        o = o + ((es[j] * rinv) @ ST) * vals[j]
    o_ref[...] = o


def _attn(q, kvs, kvg):
    grid = (NP // ABLK,)
    return pl.pallas_call(
        _attn_body,
        grid=grid,
        in_specs=[
            _row_spec(ABLK, D), _row_spec(ABLK, D),
            pl.BlockSpec((K, ABLK, D), lambda i: (0, i, 0)),
        ],
        out_specs=_row_spec(ABLK, D),
        out_shape=jax.ShapeDtypeStruct((NP, D), jnp.float32),
    )(q, kvs, kvg)


def _ff_part(x, o, wo, bo, l2s, l2b, w1, b1, w2, b2):
    x2 = x + o @ wo[...] + bo[...]
    h2 = _lnorm(x2, l2s[...], l2b[...])
    return x2 + jnp.maximum(h2 @ w1[...] + b1[...], 0.0) @ w2[...] + b2[...]


def _post_qkv_body(x_ref, o_in, wo, bo, l2s, l2b, w1, b1, w2, b2,
                   ls, lb, wq, bq, wk, bk, wv, bv,
                   x2_ref, q_ref, kv_ref):
    y = _ff_part(x_ref[...], o_in[...], wo, bo, l2s, l2b, w1, b1, w2, b2)
    x2_ref[...] = y
    q, kv = _qkv_part(y, ls, lb, wq, bq, wk, bk, wv, bv)
    q_ref[...] = q
    kv_ref[...] = kv


def _post_qkv(x, o, p, p2):
    grid = (NP // BLK,)
    shp = jax.ShapeDtypeStruct((NP, D), jnp.float32)
    shi = jax.ShapeDtypeStruct((NP, D), jnp.int32)
    return pl.pallas_call(
        _post_qkv_body,
        grid=grid,
        in_specs=[
            _row_spec(BLK, D), _row_spec(BLK, D),
            _full_spec((D, D)), _full_spec((1, D)),
            _full_spec((1, D)), _full_spec((1, D)),
            _full_spec((D, FF)), _full_spec((1, FF)),
            _full_spec((FF, D)), _full_spec((1, D)),
            _full_spec((1, D)), _full_spec((1, D)),
            _full_spec((D, D)), _full_spec((1, D)),
            _full_spec((D, D)), _full_spec((1, D)),
            _full_spec((D, D)), _full_spec((1, D)),
        ],
        out_specs=[_row_spec(BLK, D)] * 3,
        out_shape=[shp, shp, shi],
    )(x, o, p['Wo'], p['bo'].reshape(1, D),
      p['ln2_s'].reshape(1, D), p['ln2_b'].reshape(1, D),
      p['W1'], p['b1'].reshape(1, FF), p['W2'], p['b2'].reshape(1, D),
      p2['ln1_s'].reshape(1, D), p2['ln1_b'].reshape(1, D),
      p2['Wq'], p2['bq'].reshape(1, D), p2['Wk'], p2['bk'].reshape(1, D),
      p2['Wv'], p2['bv'].reshape(1, D))


def _post_dec_body(x_ref, o_in, wo, bo, l2s, l2b, w1, b1, w2, b2,
                   dw1, db1, dw2, db2, y_ref):
    y = _ff_part(x_ref[...], o_in[...], wo, bo, l2s, l2b, w1, b1, w2, b2)
    h = jnp.maximum(y @ dw1[...] + db1[...], 0.0)
    y_ref[...] = h @ dw2[...] + db2[...]


def _post_dec(x, o, p, d):
    grid = (NP // BLK,)
    return pl.pallas_call(
        _post_dec_body,
        grid=grid,
        in_specs=[
            _row_spec(BLK, D), _row_spec(BLK, D),
            _full_spec((D, D)), _full_spec((1, D)),
            _full_spec((1, D)), _full_spec((1, D)),
            _full_spec((D, FF)), _full_spec((1, FF)),
            _full_spec((FF, D)), _full_spec((1, D)),
            _full_spec((D, D)), _full_spec((1, D)),
            _full_spec((D, OUT)), _full_spec((1, OUT)),
        ],
        out_specs=_row_spec(BLK, OUT),
        out_shape=jax.ShapeDtypeStruct((NP, OUT), jnp.float32),
    )(x, o, p['Wo'], p['bo'].reshape(1, D),
      p['ln2_s'].reshape(1, D), p['ln2_b'].reshape(1, D),
      p['W1'], p['b1'].reshape(1, FF), p['W2'], p['b2'].reshape(1, D),
      d['W1'], d['b1'].reshape(1, D), d['W2'], d['b2'].reshape(1, OUT))


NB = 4
LAG = 2


def _sc_gather_body(kv_hbm, idx_hbm, kvg_hbm, idx_v, kr, *sems):
    sg, so = sems[0:NB], sems[NB:2 * NB]
    wid = lax.axis_index("s") * 2 + lax.axis_index("c")
    base = wid * IPW
    pltpu.sync_copy(idx_hbm.at[pl.ds(base, IPW)], idx_v)

    pend_g = {}
    pend_o = {}
    for i in range(NCHUNK + LAG):
        if i < NCHUNK:
            b = i % NB
            if i >= NB:
                pend_o.pop(i - NB).wait()
            ii = pl.ds(i * GCH, GCH)
            pend_g[i] = pltpu.async_copy(kv_hbm.at[idx_v.at[ii]], kr.at[b], sg[b])
        if i >= LAG:
            j = i - LAG
            b = j % NB
            pend_g.pop(j).wait()
            off = pl.ds(base + j * GCH, GCH)
            pend_o[j] = pltpu.async_copy(kr.at[b], kvg_hbm.at[off], so[b])
    for j in sorted(pend_o):
        pend_o[j].wait()


def _sc_gather(kv_all, idx_flat):
    mesh = plsc.VectorSubcoreMesh(core_axis_name="c", subcore_axis_name="s",
                                  num_cores=2, num_subcores=16)
    shp = jax.ShapeDtypeStruct((NP * K, D), jnp.int32)
    fn = pl.kernel(
        _sc_gather_body,
        out_type=shp,
        mesh=mesh,
        scratch_types=[
            pltpu.VMEM((IPW,), jnp.int32),
            pltpu.VMEM((NB, GCH, D), jnp.int32),
        ] + [pltpu.SemaphoreType.DMA] * (2 * NB),
    )
    return fn(kv_all, idx_flat)


def kernel(x, params, idx_k8):
    xp = jnp.pad(x, ((0, NP - N), (0, 0)))
    idx_flat = jnp.pad(idx_k8, ((0, NP - N), (0, 0))).T.reshape(NP * K)
    p0, p1 = params['blocks']
    h, q, kv = _enc_qkv(xp, params['enc'], p0)
    kvg = _sc_gather(kv, idx_flat).reshape(K, NP, D)
    o = _attn(q, kv, kvg)
    x2, q2, kv2 = _post_qkv(h, o, p0, p1)
    kvg2 = _sc_gather(kv2, idx_flat).reshape(K, NP, D)
    o2 = _attn(q2, kv2, kvg2)
    y = _post_dec(x2, o2, p1, params['dec'])
    return y[:N]


# speedup vs baseline: 1.2405x; 1.2405x over previous
import functools

import jax
import jax.numpy as jnp
import numpy as np
from jax import lax
from jax.experimental import pallas as pl
from jax.experimental.pallas import tpu as pltpu
from jax.experimental.pallas import tpu_sc as plsc

N = 50000
D = 128
H = 4
DH = 32
K = 8
FF = 512
OUT = 128

NW = 32
NP = 50176
PW = NP // NW
IPW = PW * K
GCH = 128
NCHUNK = IPW // GCH

BLK = 1024
ABLK = 512


def _lnorm(h, s, b):
    m = jnp.mean(h, axis=-1, keepdims=True)
    v = jnp.mean((h - m) ** 2, axis=-1, keepdims=True)
    return (h - m) * lax.rsqrt(v + 1e-5) * s + b


def _row_spec(blk, width):
    return pl.BlockSpec((blk, width), lambda i: (i, 0))


def _full_spec(shape):
    return pl.BlockSpec(shape, lambda i: tuple(0 for _ in shape))


def _pack_kv(k, v):
    kw = lax.bitcast_convert_type(k.astype(jnp.bfloat16), jnp.uint16).astype(jnp.uint32)
    vw = lax.bitcast_convert_type(v.astype(jnp.bfloat16), jnp.uint16).astype(jnp.uint32)
    return ((kw << 16) | vw).astype(jnp.int32)


def _unpack_kv(w):
    ww = lax.bitcast_convert_type(w, jnp.uint32)
    k = lax.bitcast_convert_type(ww & jnp.uint32(0xFFFF0000), jnp.float32)
    v = lax.bitcast_convert_type(ww << 16, jnp.float32)
    return k, v


def _qkv_part(h, ls, lb, wq, bq, wk, bk, wv, bv):
    hn = _lnorm(h, ls[...], lb[...])
    q = hn @ wq[...] + bq[...]
    k = hn @ wk[...] + bk[...]
    v = hn @ wv[...] + bv[...]
    return q, _pack_kv(k, v)


def _enc_qkv_body(x_ref, w1, b1, w2, b2, els, elb,
                  ls, lb, wq, bq, wk, bk, wv, bv,
                  h_ref, q_ref, kv_ref):
    h = jnp.maximum(x_ref[...] @ w1[...] + b1[...], 0.0)
    h = h @ w2[...] + b2[...]
    h = _lnorm(h, els[...], elb[...])
    h_ref[...] = h
    q, kv = _qkv_part(h, ls, lb, wq, bq, wk, bk, wv, bv)
    q_ref[...] = q
    kv_ref[...] = kv


def _enc_qkv(x, e, p):
    grid = (NP // BLK,)
    shp = jax.ShapeDtypeStruct((NP, D), jnp.float32)
    shi = jax.ShapeDtypeStruct((NP, D), jnp.int32)
    return pl.pallas_call(
        _enc_qkv_body,
        grid=grid,
        in_specs=[
            _row_spec(BLK, D),
            _full_spec((D, D)), _full_spec((1, D)),
            _full_spec((D, D)), _full_spec((1, D)),
            _full_spec((1, D)), _full_spec((1, D)),
            _full_spec((1, D)), _full_spec((1, D)),
            _full_spec((D, D)), _full_spec((1, D)),
            _full_spec((D, D)), _full_spec((1, D)),
            _full_spec((D, D)), _full_spec((1, D)),
        ],
        out_specs=[_row_spec(BLK, D)] * 3,
        out_shape=[shp, shp, shi],
    )(x, e['W1'], e['b1'].reshape(1, D), e['W2'], e['b2'].reshape(1, D),
      e['ln_s'].reshape(1, D), e['ln_b'].reshape(1, D),
      p['ln1_s'].reshape(1, D), p['ln1_b'].reshape(1, D),
      p['Wq'], p['bq'].reshape(1, D), p['Wk'], p['bk'].reshape(1, D),
      p['Wv'], p['bv'].reshape(1, D))


def _attn_body(q_ref, kvs_ref, kvg_ref, o_ref):
    q = q_ref[...]
    r = lax.broadcasted_iota(jnp.int32, (D, H), 0) // DH
    c = lax.broadcasted_iota(jnp.int32, (D, H), 1)
    S = (r == c).astype(jnp.float32)
    r2 = lax.broadcasted_iota(jnp.int32, (H, D), 0)
    c2 = lax.broadcasted_iota(jnp.int32, (H, D), 1) // DH
    ST = (r2 == c2).astype(jnp.float32)
    scale = np.float32(1.0 / np.sqrt(DH))
    logit = []
    vals = []
    for j in range(K + 1):
        kj, vj = _unpack_kv(kvs_ref[...] if j == 0 else kvg_ref[j - 1])
        logit.append(((q * kj) @ S) * scale)
        vals.append(vj)
    m = logit[0]
    for x in logit[1:]:
        m = jnp.maximum(m, x)
    es = [jnp.exp(x - m) for x in logit]
    ssum = es[0]
    for e in es[1:]:
        ssum = ssum + e
    rinv = 1.0 / ssum
    o = ((es[0] * rinv) @ ST) * vals[0]
    for j in range(1, K + 1):
        o = o + ((es[j] * rinv) @ ST) * vals[j]
    o_ref[...] = o


def _attn(q, kvs, kvg):
    grid = (NP // ABLK,)
    return pl.pallas_call(
        _attn_body,
        grid=grid,
        in_specs=[
            _row_spec(ABLK, D), _row_spec(ABLK, D),
            pl.BlockSpec((K, ABLK, D), lambda i: (0, i, 0)),
        ],
        out_specs=_row_spec(ABLK, D),
        out_shape=jax.ShapeDtypeStruct((NP, D), jnp.float32),
    )(q, kvs, kvg)


def _ff_part(x, o, wo, bo, l2s, l2b, w1, b1, w2, b2):
    x2 = x + o @ wo[...] + bo[...]
    h2 = _lnorm(x2, l2s[...], l2b[...])
    return x2 + jnp.maximum(h2 @ w1[...] + b1[...], 0.0) @ w2[...] + b2[...]


def _post_qkv_body(x_ref, o_in, wo, bo, l2s, l2b, w1, b1, w2, b2,
                   ls, lb, wq, bq, wk, bk, wv, bv,
                   x2_ref, q_ref, kv_ref):
    y = _ff_part(x_ref[...], o_in[...], wo, bo, l2s, l2b, w1, b1, w2, b2)
    x2_ref[...] = y
    q, kv = _qkv_part(y, ls, lb, wq, bq, wk, bk, wv, bv)
    q_ref[...] = q
    kv_ref[...] = kv


def _post_qkv(x, o, p, p2):
    grid = (NP // BLK,)
    shp = jax.ShapeDtypeStruct((NP, D), jnp.float32)
    shi = jax.ShapeDtypeStruct((NP, D), jnp.int32)
    return pl.pallas_call(
        _post_qkv_body,
        grid=grid,
        in_specs=[
            _row_spec(BLK, D), _row_spec(BLK, D),
            _full_spec((D, D)), _full_spec((1, D)),
            _full_spec((1, D)), _full_spec((1, D)),
            _full_spec((D, FF)), _full_spec((1, FF)),
            _full_spec((FF, D)), _full_spec((1, D)),
            _full_spec((1, D)), _full_spec((1, D)),
            _full_spec((D, D)), _full_spec((1, D)),
            _full_spec((D, D)), _full_spec((1, D)),
            _full_spec((D, D)), _full_spec((1, D)),
        ],
        out_specs=[_row_spec(BLK, D)] * 3,
        out_shape=[shp, shp, shi],
    )(x, o, p['Wo'], p['bo'].reshape(1, D),
      p['ln2_s'].reshape(1, D), p['ln2_b'].reshape(1, D),
      p['W1'], p['b1'].reshape(1, FF), p['W2'], p['b2'].reshape(1, D),
      p2['ln1_s'].reshape(1, D), p2['ln1_b'].reshape(1, D),
      p2['Wq'], p2['bq'].reshape(1, D), p2['Wk'], p2['bk'].reshape(1, D),
      p2['Wv'], p2['bv'].reshape(1, D))


def _post_dec_body(x_ref, o_in, wo, bo, l2s, l2b, w1, b1, w2, b2,
                   dw1, db1, dw2, db2, y_ref):
    y = _ff_part(x_ref[...], o_in[...], wo, bo, l2s, l2b, w1, b1, w2, b2)
    h = jnp.maximum(y @ dw1[...] + db1[...], 0.0)
    y_ref[...] = h @ dw2[...] + db2[...]


def _post_dec(x, o, p, d):
    grid = (NP // BLK,)
    return pl.pallas_call(
        _post_dec_body,
        grid=grid,
        in_specs=[
            _row_spec(BLK, D), _row_spec(BLK, D),
            _full_spec((D, D)), _full_spec((1, D)),
            _full_spec((1, D)), _full_spec((1, D)),
            _full_spec((D, FF)), _full_spec((1, FF)),
            _full_spec((FF, D)), _full_spec((1, D)),
            _full_spec((D, D)), _full_spec((1, D)),
            _full_spec((D, OUT)), _full_spec((1, OUT)),
        ],
        out_specs=_row_spec(BLK, OUT),
        out_shape=jax.ShapeDtypeStruct((NP, OUT), jnp.float32),
    )(x, o, p['Wo'], p['bo'].reshape(1, D),
      p['ln2_s'].reshape(1, D), p['ln2_b'].reshape(1, D),
      p['W1'], p['b1'].reshape(1, FF), p['W2'], p['b2'].reshape(1, D),
      d['W1'], d['b1'].reshape(1, D), d['W2'], d['b2'].reshape(1, OUT))


NB = 4
LAG = 2


def _sc_gather_body(kv_hbm, idx_hbm, kvg_hbm, idx_v, kr, *sems):
    sg, so = sems[0:NB], sems[NB:2 * NB]
    wid = lax.axis_index("s") * 2 + lax.axis_index("c")
    base = wid * IPW
    pltpu.sync_copy(idx_hbm.at[pl.ds(base, IPW)], idx_v)

    pend_g = {}
    pend_o = {}
    for i in range(NCHUNK + LAG):
        if i < NCHUNK:
            b = i % NB
            if i >= NB:
                pend_o.pop(i - NB).wait()
            ii = pl.ds(i * GCH, GCH)
            pend_g[i] = pltpu.async_copy(kv_hbm.at[idx_v.at[ii]], kr.at[b], sg[b])
        if i >= LAG:
            j = i - LAG
            b = j % NB
            pend_g.pop(j).wait()
            off = pl.ds(base + j * GCH, GCH)
            pend_o[j] = pltpu.async_copy(kr.at[b], kvg_hbm.at[off], so[b])
    for j in sorted(pend_o):
        pend_o[j].wait()


def _sc_gather(kv_all, idx_flat):
    mesh = plsc.VectorSubcoreMesh(core_axis_name="c", subcore_axis_name="s",
                                  num_cores=2, num_subcores=16)
    shp = jax.ShapeDtypeStruct((NP * K, D), jnp.int32)
    fn = pl.kernel(
        _sc_gather_body,
        out_type=shp,
        mesh=mesh,
        scratch_types=[
            pltpu.VMEM((IPW,), jnp.int32),
            pltpu.VMEM((NB, GCH, D), jnp.int32),
        ] + [pltpu.SemaphoreType.DMA] * (2 * NB),
    )
    return fn(kv_all, idx_flat)


def kernel(x, params, idx_k8):
    xp = jnp.pad(x, ((0, NP - N), (0, 0)))
    idx_flat = jnp.pad(idx_k8, ((0, NP - N), (0, 0))).T.reshape(NP * K)
    p0, p1 = params['blocks']
    h, q, kv = _enc_qkv(xp, params['enc'], p0)
    kvg = _sc_gather(kv, idx_flat).reshape(K, NP, D)
    o = _attn(q, kv, kvg)
    x2, q2, kv2 = _post_qkv(h, o, p0, p1)
    kvg2 = _sc_gather(kv2, idx_flat).reshape(K, NP, D)
    o2 = _attn(q2, kv2, kvg2)
    y = _post_dec(x2, o2, p1, params['dec'])
    return y[:N]


# speedup vs baseline: 1.2579x; 1.0140x over previous
import functools

import jax
import jax.numpy as jnp
import numpy as np
from jax import lax
from jax.experimental import pallas as pl
from jax.experimental.pallas import tpu as pltpu
from jax.experimental.pallas import tpu_sc as plsc

N = 50000
D = 128
H = 4
DH = 32
K = 8
FF = 512
OUT = 128

NW = 32
NP = 50176
PW = NP // NW
IPW = PW * K
GCH = 128
NCHUNK = IPW // GCH

BLK = 1024
ABLK = 512


def _lnorm(h, s, b):
    m = jnp.mean(h, axis=-1, keepdims=True)
    v = jnp.mean((h - m) ** 2, axis=-1, keepdims=True)
    return (h - m) * lax.rsqrt(v + 1e-5) * s + b


def _row_spec(blk, width):
    return pl.BlockSpec((blk, width), lambda i: (i, 0))


def _full_spec(shape):
    return pl.BlockSpec(shape, lambda i: tuple(0 for _ in shape))


def _pack_kv(k, v):
    kw = lax.bitcast_convert_type(k.astype(jnp.bfloat16), jnp.uint16).astype(jnp.uint32)
    vw = lax.bitcast_convert_type(v.astype(jnp.bfloat16), jnp.uint16).astype(jnp.uint32)
    return ((kw << 16) | vw).astype(jnp.int32)


def _unpack_kv(w):
    ww = lax.bitcast_convert_type(w, jnp.uint32)
    k = lax.bitcast_convert_type(ww & jnp.uint32(0xFFFF0000), jnp.float32)
    v = lax.bitcast_convert_type(ww << 16, jnp.float32)
    return k, v


def _qkv_part(h, ls, lb, wq, bq, wk, bk, wv, bv):
    hn = _lnorm(h, ls[...], lb[...])
    q = hn @ wq[...] + bq[...]
    k = hn @ wk[...] + bk[...]
    v = hn @ wv[...] + bv[...]
    return q, _pack_kv(k, v)


def _enc_qkv_body(x_ref, w1, b1, w2, b2, els, elb,
                  ls, lb, wq, bq, wk, bk, wv, bv,
                  h_ref, q_ref, kv_ref):
    h = jnp.maximum(x_ref[...] @ w1[...] + b1[...], 0.0)
    h = h @ w2[...] + b2[...]
    h = _lnorm(h, els[...], elb[...])
    h_ref[...] = h
    q, kv = _qkv_part(h, ls, lb, wq, bq, wk, bk, wv, bv)
    q_ref[...] = q
    kv_ref[...] = kv


def _enc_qkv(x, e, p):
    grid = (NP // BLK,)
    shp = jax.ShapeDtypeStruct((NP, D), jnp.float32)
    shi = jax.ShapeDtypeStruct((NP, D), jnp.int32)
    return pl.pallas_call(
        _enc_qkv_body,
        grid=grid,
        in_specs=[
            _row_spec(BLK, D),
            _full_spec((D, D)), _full_spec((1, D)),
            _full_spec((D, D)), _full_spec((1, D)),
            _full_spec((1, D)), _full_spec((1, D)),
            _full_spec((1, D)), _full_spec((1, D)),
            _full_spec((D, D)), _full_spec((1, D)),
            _full_spec((D, D)), _full_spec((1, D)),
            _full_spec((D, D)), _full_spec((1, D)),
        ],
        out_specs=[_row_spec(BLK, D)] * 3,
        out_shape=[shp, shp, shi],
    )(x, e['W1'], e['b1'].reshape(1, D), e['W2'], e['b2'].reshape(1, D),
      e['ln_s'].reshape(1, D), e['ln_b'].reshape(1, D),
      p['ln1_s'].reshape(1, D), p['ln1_b'].reshape(1, D),
      p['Wq'], p['bq'].reshape(1, D), p['Wk'], p['bk'].reshape(1, D),
      p['Wv'], p['bv'].reshape(1, D))


def _attn_body(q_ref, kvs_ref, kvg_ref, o_ref):
    q = q_ref[...]
    r = lax.broadcasted_iota(jnp.int32, (D, H), 0) // DH
    c = lax.broadcasted_iota(jnp.int32, (D, H), 1)
    S = (r == c).astype(jnp.float32)
    r2 = lax.broadcasted_iota(jnp.int32, (H, D), 0)
    c2 = lax.broadcasted_iota(jnp.int32, (H, D), 1) // DH
    ST = (r2 == c2).astype(jnp.float32)
    scale = np.float32(1.0 / np.sqrt(DH))
    logit = []
    vals = []
    for j in range(K + 1):
        kj, vj = _unpack_kv(kvs_ref[...] if j == 0 else kvg_ref[j - 1])
        logit.append(((q * kj) @ S) * scale)
        vals.append(vj)
    m = logit[0]
    for x in logit[1:]:
        m = jnp.maximum(m, x)
    es = [jnp.exp(x - m) for x in logit]
    ssum = es[0]
    for e in es[1:]:
        ssum = ssum + e
    rinv = 1.0 / ssum
    o = ((es[0] * rinv) @ ST) * vals[0]
    for j in range(1, K + 1):
        o = o + ((es[j] * rinv) @ ST) * vals[j]
    o_ref[...] = o


def _attn(q, kvs, kvg, nrows=NP, row0=0):
    grid = (nrows // ABLK,)
    blk0 = row0 // ABLK
    return pl.pallas_call(
        _attn_body,
        grid=grid,
        in_specs=[
            pl.BlockSpec((ABLK, D), lambda i: (i + blk0, 0)),
            pl.BlockSpec((ABLK, D), lambda i: (i + blk0, 0)),
            pl.BlockSpec((K, ABLK, D), lambda i: (0, i, 0)),
        ],
        out_specs=_row_spec(ABLK, D),
        out_shape=jax.ShapeDtypeStruct((nrows, D), jnp.float32),
    )(q, kvs, kvg)


def _ff_part(x, o, wo, bo, l2s, l2b, w1, b1, w2, b2):
    x2 = x + o @ wo[...] + bo[...]
    h2 = _lnorm(x2, l2s[...], l2b[...])
    return x2 + jnp.maximum(h2 @ w1[...] + b1[...], 0.0) @ w2[...] + b2[...]


def _post_qkv_body(x_ref, o_in, wo, bo, l2s, l2b, w1, b1, w2, b2,
                   ls, lb, wq, bq, wk, bk, wv, bv,
                   x2_ref, q_ref, kv_ref):
    y = _ff_part(x_ref[...], o_in[...], wo, bo, l2s, l2b, w1, b1, w2, b2)
    x2_ref[...] = y
    q, kv = _qkv_part(y, ls, lb, wq, bq, wk, bk, wv, bv)
    q_ref[...] = q
    kv_ref[...] = kv


def _post_qkv(x, o, p, p2):
    grid = (NP // BLK,)
    shp = jax.ShapeDtypeStruct((NP, D), jnp.float32)
    shi = jax.ShapeDtypeStruct((NP, D), jnp.int32)
    return pl.pallas_call(
        _post_qkv_body,
        grid=grid,
        in_specs=[
            _row_spec(BLK, D), _row_spec(BLK, D),
            _full_spec((D, D)), _full_spec((1, D)),
            _full_spec((1, D)), _full_spec((1, D)),
            _full_spec((D, FF)), _full_spec((1, FF)),
            _full_spec((FF, D)), _full_spec((1, D)),
            _full_spec((1, D)), _full_spec((1, D)),
            _full_spec((D, D)), _full_spec((1, D)),
            _full_spec((D, D)), _full_spec((1, D)),
            _full_spec((D, D)), _full_spec((1, D)),
        ],
        out_specs=[_row_spec(BLK, D)] * 3,
        out_shape=[shp, shp, shi],
    )(x, o, p['Wo'], p['bo'].reshape(1, D),
      p['ln2_s'].reshape(1, D), p['ln2_b'].reshape(1, D),
      p['W1'], p['b1'].reshape(1, FF), p['W2'], p['b2'].reshape(1, D),
      p2['ln1_s'].reshape(1, D), p2['ln1_b'].reshape(1, D),
      p2['Wq'], p2['bq'].reshape(1, D), p2['Wk'], p2['bk'].reshape(1, D),
      p2['Wv'], p2['bv'].reshape(1, D))


def _post_dec_body(x_ref, o_in, wo, bo, l2s, l2b, w1, b1, w2, b2,
                   dw1, db1, dw2, db2, y_ref):
    y = _ff_part(x_ref[...], o_in[...], wo, bo, l2s, l2b, w1, b1, w2, b2)
    h = jnp.maximum(y @ dw1[...] + db1[...], 0.0)
    y_ref[...] = h @ dw2[...] + db2[...]


def _post_dec(x, o, p, d):
    grid = (NP // BLK,)
    return pl.pallas_call(
        _post_dec_body,
        grid=grid,
        in_specs=[
            _row_spec(BLK, D), _row_spec(BLK, D),
            _full_spec((D, D)), _full_spec((1, D)),
            _full_spec((1, D)), _full_spec((1, D)),
            _full_spec((D, FF)), _full_spec((1, FF)),
            _full_spec((FF, D)), _full_spec((1, D)),
            _full_spec((D, D)), _full_spec((1, D)),
            _full_spec((D, OUT)), _full_spec((1, OUT)),
        ],
        out_specs=_row_spec(BLK, OUT),
        out_shape=jax.ShapeDtypeStruct((NP, OUT), jnp.float32),
    )(x, o, p['Wo'], p['bo'].reshape(1, D),
      p['ln2_s'].reshape(1, D), p['ln2_b'].reshape(1, D),
      p['W1'], p['b1'].reshape(1, FF), p['W2'], p['b2'].reshape(1, D),
      d['W1'], d['b1'].reshape(1, D), d['W2'], d['b2'].reshape(1, OUT))


NB = 4
LAG = 2


def _sc_gather_body(nidx, kv_hbm, idx_hbm, kvg_hbm, idx_v, kr, *sems):
    ipw = nidx // NW
    nchunk = ipw // GCH
    sg, so = sems[0:NB], sems[NB:2 * NB]
    wid = lax.axis_index("s") * 2 + lax.axis_index("c")
    base = wid * ipw
    pltpu.sync_copy(idx_hbm.at[pl.ds(base, ipw)], idx_v)

    pend_g = {}
    pend_o = {}
    for i in range(nchunk + LAG):
        if i < nchunk:
            b = i % NB
            if i >= NB:
                pend_o.pop(i - NB).wait()
            ii = pl.ds(i * GCH, GCH)
            pend_g[i] = pltpu.async_copy(kv_hbm.at[idx_v.at[ii]], kr.at[b], sg[b])
        if i >= LAG:
            j = i - LAG
            b = j % NB
            pend_g.pop(j).wait()
            off = pl.ds(base + j * GCH, GCH)
            pend_o[j] = pltpu.async_copy(kr.at[b], kvg_hbm.at[off], so[b])
    for j in sorted(pend_o):
        pend_o[j].wait()


def _sc_gather(kv_all, idx_flat):
    nidx = idx_flat.shape[0]
    mesh = plsc.VectorSubcoreMesh(core_axis_name="c", subcore_axis_name="s",
                                  num_cores=2, num_subcores=16)
    shp = jax.ShapeDtypeStruct((nidx, D), jnp.int32)
    fn = pl.kernel(
        functools.partial(_sc_gather_body, nidx),
        out_type=shp,
        mesh=mesh,
        scratch_types=[
            pltpu.VMEM((nidx // NW,), jnp.int32),
            pltpu.VMEM((NB, GCH, D), jnp.int32),
        ] + [pltpu.SemaphoreType.DMA] * (2 * NB),
    )
    return fn(kv_all, idx_flat)


def kernel(x, params, idx_k8):
    xp = jnp.pad(x, ((0, NP - N), (0, 0)))
    HNP = NP // 2
    idx2 = jnp.pad(idx_k8, ((0, NP - N), (0, 0))).T
    idx_a = idx2[:, :HNP].reshape(K * HNP)
    idx_b = idx2[:, HNP:].reshape(K * HNP)
    p0, p1 = params['blocks']

    def attn_stage(q, kv):
        kvga = _sc_gather(kv, idx_a).reshape(K, HNP, D)
        kvgb = _sc_gather(kv, idx_b).reshape(K, HNP, D)
        oa = _attn(q, kv, kvga, HNP, 0)
        ob = _attn(q, kv, kvgb, HNP, HNP)
        return jnp.concatenate([oa, ob], axis=0)

    h, q, kv = _enc_qkv(xp, params['enc'], p0)
    o = attn_stage(q, kv)
    x2, q2, kv2 = _post_qkv(h, o, p0, p1)
    o2 = attn_stage(q2, kv2)
    y = _post_dec(x2, o2, p1, params['dec'])
    return y[:N]
